# baseline (speedup 1.0000x reference)
import functools

import jax
import jax.numpy as jnp
from jax.experimental import pallas as pl
from jax.experimental.pallas import tpu as pltpu
from jax.experimental.pallas import tpu_sc as plsc

_T, _D, _E, _TOPK, _NG, _TG, _DFF, _DFFS = 2048, 1024, 16, 2, 4, 2, 512, 512
_GS = _E // _NG
_ROUTE_SCALE = 2.5
_C = 512
_B = 128
_S = _E * _C
_XG_ROWS = _S + _B
_NC, _NS = 2, 16
_NW = _NC * _NS


def _router_body(x_ref, gw_ref, eb_ref, meta_ref, cnt_ref):
    x = x_ref[...]
    logits = jax.lax.dot_general(
        x, gw_ref[...], (((1,), (1,)), ((), ())),
        preferred_element_type=jnp.float32)
    scores = jax.nn.sigmoid(logits)
    sfc = scores + eb_ref[...]
    gs_cols = []
    for g in range(_NG):
        c = [sfc[:, g * _GS + i:g * _GS + i + 1] for i in range(_GS)]
        best = None
        for i in range(_GS):
            for j in range(i + 1, _GS):
                s = c[i] + c[j]
                best = s if best is None else jnp.maximum(best, s)
        gs_cols.append(best)
    gs = jnp.concatenate(gs_cols, axis=1)
    gidx = jax.lax.broadcasted_iota(jnp.int32, (_T, _NG), 1)
    grank = jnp.zeros((_T, _NG), jnp.float32)
    for j in range(_NG):
        gj = gs[:, j:j + 1]
        grank += jnp.where((gj > gs) | ((gj == gs) & (j < gidx)), 1.0, 0.0)
    gsel = (grank < _TG).astype(jnp.float32)
    emask = jnp.concatenate(
        [gsel[:, e // _GS:e // _GS + 1] for e in range(_E)], axis=1)
    tmp = sfc * emask
    eidx = jax.lax.broadcasted_iota(jnp.int32, (_T, _E), 1)
    erank = jnp.zeros((_T, _E), jnp.float32)
    for j in range(_E):
        vj = tmp[:, j:j + 1]
        erank += jnp.where((vj > tmp) | ((vj == tmp) & (j < eidx)), 1.0, 0.0)
    sel = jnp.where(erank < _TOPK, 1.0, 0.0)
    w = scores * sel
    denom = jnp.sum(w, axis=1, keepdims=True) + 1e-20
    wfull = w * (_ROUTE_SCALE / denom)
    selb = sel.astype(jnp.bfloat16)
    riota = jax.lax.broadcasted_iota(jnp.int32, (_T, 1), 0)
    ciota = jax.lax.broadcasted_iota(jnp.int32, (1, _T), 1)
    ltri = (riota >= ciota).astype(jnp.bfloat16)
    pos = jax.lax.dot_general(
        ltri, selb, (((1,), (0,)), ((), ())),
        preferred_element_type=jnp.float32)
    cnt_ref[...] = pos[_T - 1:_T, :]
    eidx_f = eidx.astype(jnp.float32)
    slot = eidx_f * _C + (pos - 1.0)
    slot = jnp.where(pos - 1.0 < _C, slot, float(_S))
    r16 = jax.lax.broadcasted_iota(jnp.int32, (_E, _E), 0)
    c16 = jax.lax.broadcasted_iota(jnp.int32, (_E, _E), 1)
    ltri16 = (r16 <= c16).astype(jnp.bfloat16)
    cl = jax.lax.dot_general(
        selb, ltri16, (((1,), (0,)), ((), ())),
        preferred_element_type=jnp.float32)
    low = sel * jnp.where(cl == 1.0, 1.0, 0.0)
    high = sel * jnp.where(cl == 2.0, 1.0, 0.0)
    dst0 = jnp.sum(slot * low, axis=1, keepdims=True)
    dst1 = jnp.sum(slot * high, axis=1, keepdims=True)
    w0 = jnp.sum(wfull * low, axis=1, keepdims=True)
    w1 = jnp.sum(wfull * high, axis=1, keepdims=True)
    meta_ref[:, 0:4] = jnp.concatenate([dst0, dst1, w0, w1], axis=1)


def _dot_t(a, b):
    return jax.lax.dot_general(
        a, b, (((1,), (1,)), ((), ())), preferred_element_type=jnp.float32)


def _group_body(cnt_ref, xg_ref, w1_ref, w3_ref, w2_ref, yg_ref):
    cb = pl.program_id(1)

    @pl.when(cb * _B < cnt_ref[0, pl.program_id(0)])
    def _():
        xgb = xg_ref[...]
        g = _dot_t(xgb, w1_ref[0].astype(jnp.bfloat16))
        u = _dot_t(xgb, w3_ref[0].astype(jnp.bfloat16))
        h = (g * jax.nn.sigmoid(g) * u).astype(jnp.bfloat16)
        yg_ref[...] = _dot_t(h, w2_ref[0].astype(jnp.bfloat16))


def _shared_body(xb_ref, sw1_ref, sw3_ref, sw2_ref, o_ref):
    xb = xb_ref[...]
    g = _dot_t(xb, sw1_ref[...].astype(jnp.bfloat16))
    u = _dot_t(xb, sw3_ref[...].astype(jnp.bfloat16))
    h = (g * jax.nn.sigmoid(g) * u).astype(jnp.bfloat16)
    o_ref[...] = _dot_t(h, sw2_ref[...].astype(jnp.bfloat16))


def _dispatch_body(xb_hbm, idx_hbm, xg_hbm, i_v, rows_v, semi, semr):
    c = jax.lax.axis_index("c")
    s = jax.lax.axis_index("s")
    wid = c * _NS + s
    j0 = wid * (2 * _T // _NW)
    r = j0 // _T
    t0 = j0 % _T
    cps = []
    for cc in range(2):
        tb = t0 + cc * 64
        ldi = pltpu.async_copy(idx_hbm.at[r, pl.ds(tb, 64)], i_v.at[cc], semi)
        ldr = pltpu.async_copy(xb_hbm.at[pl.ds(tb, 64)], rows_v.at[cc], semr)
        cps.append((ldi, ldr))
    for cc in range(2):
        ldi, ldr = cps[cc]
        ldi.wait()
        ldr.wait()
        pltpu.sync_copy(rows_v.at[cc], xg_hbm.at[i_v.at[cc]])


def _combine_body(yg_hbm, idx_hbm, wexp_hbm, sh_hbm, out_hbm,
                  i_v, w_v, sh_v, y0_v, y1_v, sems):
    c = jax.lax.axis_index("c")
    s = jax.lax.axis_index("s")
    wid = c * _NS + s
    t0 = wid * (_T // _NW)
    nch = _T // _NW // 16

    def issue(cc):
        b = cc % 2
        tb = t0 + cc * 16
        pltpu.sync_copy(idx_hbm.at[0, pl.ds(tb, 16)], i_v.at[b, 0])
        pltpu.sync_copy(idx_hbm.at[1, pl.ds(tb, 16)], i_v.at[b, 1])
        pltpu.sync_copy(wexp_hbm.at[0, pl.ds(tb, 16)], w_v.at[b, 0])
        pltpu.sync_copy(wexp_hbm.at[1, pl.ds(tb, 16)], w_v.at[b, 1])
        pltpu.sync_copy(sh_hbm.at[pl.ds(tb, 16)], sh_v.at[b])
        g0 = pltpu.async_copy(yg_hbm.at[i_v.at[b, 0]], y0_v.at[b], sems.at[2 * b])
        g1 = pltpu.async_copy(yg_hbm.at[i_v.at[b, 1]], y1_v.at[b], sems.at[2 * b + 1])
        return g0, g1

    pend = issue(0)
    for cc in range(nch):
        nxt = issue(cc + 1) if cc + 1 < nch else None
        b = cc % 2
        pend[0].wait()
        pend[1].wait()

        def tok_body(i, carry):
            w0s = w_v[b, 0, i]
            w1s = w_v[b, 1, i]

            def col_body(k, carry2):
                sl = pl.ds(k * 16, 16)
                sh_v[b, i, sl] = (sh_v[b, i, sl] + w0s * y0_v[b, i, sl]
                                  + w1s * y1_v[b, i, sl])
                return carry2

            return jax.lax.fori_loop(0, _D // 16, col_body, carry, unroll=8)

        jax.lax.fori_loop(0, 16, tok_body, 0)
        pltpu.sync_copy(sh_v.at[b], out_hbm.at[pl.ds(t0 + cc * 16, 16)])
        pend = nxt


def _sc_mesh():
    return plsc.VectorSubcoreMesh(
        core_axis_name="c", subcore_axis_name="s",
        num_cores=_NC, num_subcores=_NS)


def _sc_dispatch(xb32, idx):
    return pl.kernel(
        _dispatch_body,
        out_type=jax.ShapeDtypeStruct((_XG_ROWS, _D // 2), jnp.int32),
        mesh=_sc_mesh(),
        scratch_types=[pltpu.VMEM((2, 64), jnp.int32),
                       pltpu.VMEM((2, 64, _D // 2), jnp.int32),
                       pltpu.SemaphoreType.DMA,
                       pltpu.SemaphoreType.DMA],
    )(xb32, idx)


def _sc_combine(yg, idx, wexp, shared):
    return pl.kernel(
        _combine_body,
        out_type=jax.ShapeDtypeStruct((_T, _D), jnp.float32),
        mesh=_sc_mesh(),
        scratch_types=[pltpu.VMEM((2, 2, 16), jnp.int32),
                       pltpu.VMEM((2, 2, 16, 16), jnp.float32),
                       pltpu.VMEM((2, 16, _D), jnp.float32),
                       pltpu.VMEM((2, 16, _D), jnp.float32),
                       pltpu.VMEM((2, 16, _D), jnp.float32),
                       pltpu.SemaphoreType.DMA((4,))],
    )(yg, idx, wexp, shared)


def kernel(hidden_states, gate_w, expert_bias, w1, w3, w2, sw1, sw3, sw2):
    x = hidden_states.reshape(_T, _D)
    eb = expert_bias.reshape(1, _E)
    meta, cnts = pl.pallas_call(
        _router_body,
        out_shape=(jax.ShapeDtypeStruct((_T, 128), jnp.float32),
                   jax.ShapeDtypeStruct((1, _E), jnp.float32)),
    )(x, gate_w, eb)
    idx = jnp.transpose(meta[:, 0:2]).astype(jnp.int32)
    wexp = jnp.broadcast_to(
        jnp.transpose(meta[:, 2:4])[:, :, None], (2, _T, 16))
    counts = cnts.astype(jnp.int32)
    xb = x.astype(jnp.bfloat16)

    xb32 = jax.lax.bitcast_convert_type(
        xb.reshape(_T, _D // 2, 2), jnp.int32)
    xg32 = _sc_dispatch(xb32, idx)
    xg = jax.lax.bitcast_convert_type(
        xg32, jnp.bfloat16).reshape(_XG_ROWS, _D)

    shared = pl.pallas_call(
        _shared_body,
        out_shape=jax.ShapeDtypeStruct((_T, _D), jnp.float32),
    )(xb, sw1, sw3, sw2)

    yg = pl.pallas_call(
        _group_body,
        grid=(_E, _C // _B),
        in_specs=[
            pl.BlockSpec(memory_space=pltpu.SMEM),
            pl.BlockSpec((_B, _D), lambda e, cb: (e * (_C // _B) + cb, 0)),
            pl.BlockSpec((1, _DFF, _D), lambda e, cb: (e, 0, 0)),
            pl.BlockSpec((1, _DFF, _D), lambda e, cb: (e, 0, 0)),
            pl.BlockSpec((1, _D, _DFF), lambda e, cb: (e, 0, 0)),
        ],
        out_specs=pl.BlockSpec((_B, _D), lambda e, cb: (e * (_C // _B) + cb, 0)),
        out_shape=jax.ShapeDtypeStruct((_XG_ROWS, _D), jnp.float32),
    )(counts, xg, w1, w3, w2)

    out = _sc_combine(yg, idx, wexp, shared)
    return out


# speedup vs baseline: 2.0308x; 2.0308x over previous
import functools

import jax
import jax.numpy as jnp
from jax.experimental import pallas as pl
from jax.experimental.pallas import tpu as pltpu
from jax.experimental.pallas import tpu_sc as plsc

_T, _D, _E, _TOPK, _NG, _TG, _DFF, _DFFS = 2048, 1024, 16, 2, 4, 2, 512, 512
_GS = _E // _NG
_ROUTE_SCALE = 2.5
_C = 512
_B = 128
_S = _E * _C
_XG_ROWS = _S + _B
_NC, _NS = 2, 16
_NW = _NC * _NS


def _router_body(x_ref, gw_ref, eb_ref, meta_ref, cnt_ref):
    x = x_ref[...]
    logits = jax.lax.dot_general(
        x, gw_ref[...], (((1,), (1,)), ((), ())),
        preferred_element_type=jnp.float32)
    scores = jax.nn.sigmoid(logits)
    sfc = scores + eb_ref[...]
    gs_cols = []
    for g in range(_NG):
        c = [sfc[:, g * _GS + i:g * _GS + i + 1] for i in range(_GS)]
        best = None
        for i in range(_GS):
            for j in range(i + 1, _GS):
                s = c[i] + c[j]
                best = s if best is None else jnp.maximum(best, s)
        gs_cols.append(best)
    gs = jnp.concatenate(gs_cols, axis=1)
    gidx = jax.lax.broadcasted_iota(jnp.int32, (_T, _NG), 1)
    grank = jnp.zeros((_T, _NG), jnp.float32)
    for j in range(_NG):
        gj = gs[:, j:j + 1]
        grank += jnp.where((gj > gs) | ((gj == gs) & (j < gidx)), 1.0, 0.0)
    gsel = (grank < _TG).astype(jnp.float32)
    emask = jnp.concatenate(
        [gsel[:, e // _GS:e // _GS + 1] for e in range(_E)], axis=1)
    tmp = sfc * emask
    eidx = jax.lax.broadcasted_iota(jnp.int32, (_T, _E), 1)
    erank = jnp.zeros((_T, _E), jnp.float32)
    for j in range(_E):
        vj = tmp[:, j:j + 1]
        erank += jnp.where((vj > tmp) | ((vj == tmp) & (j < eidx)), 1.0, 0.0)
    sel = jnp.where(erank < _TOPK, 1.0, 0.0)
    w = scores * sel
    denom = jnp.sum(w, axis=1, keepdims=True) + 1e-20
    wfull = w * (_ROUTE_SCALE / denom)
    selb = sel.astype(jnp.bfloat16)
    riota = jax.lax.broadcasted_iota(jnp.int32, (_T, 1), 0)
    ciota = jax.lax.broadcasted_iota(jnp.int32, (1, _T), 1)
    ltri = (riota >= ciota).astype(jnp.bfloat16)
    pos = jax.lax.dot_general(
        ltri, selb, (((1,), (0,)), ((), ())),
        preferred_element_type=jnp.float32)
    cnt_ref[...] = pos[_T - 1:_T, :]
    eidx_f = eidx.astype(jnp.float32)
    slot = eidx_f * _C + (pos - 1.0)
    slot = jnp.where(pos - 1.0 < _C, slot, float(_S))
    r16 = jax.lax.broadcasted_iota(jnp.int32, (_E, _E), 0)
    c16 = jax.lax.broadcasted_iota(jnp.int32, (_E, _E), 1)
    ltri16 = (r16 <= c16).astype(jnp.bfloat16)
    cl = jax.lax.dot_general(
        selb, ltri16, (((1,), (0,)), ((), ())),
        preferred_element_type=jnp.float32)
    low = sel * jnp.where(cl == 1.0, 1.0, 0.0)
    high = sel * jnp.where(cl == 2.0, 1.0, 0.0)
    dst0 = jnp.sum(slot * low, axis=1, keepdims=True)
    dst1 = jnp.sum(slot * high, axis=1, keepdims=True)
    w0 = jnp.sum(wfull * low, axis=1, keepdims=True)
    w1 = jnp.sum(wfull * high, axis=1, keepdims=True)
    meta_ref[:, 0:4] = jnp.concatenate([dst0, dst1, w0, w1], axis=1)


def _dot_t(a, b):
    return jax.lax.dot_general(
        a, b, (((1,), (1,)), ((), ())), preferred_element_type=jnp.float32)


def _group_body(cnt_ref, xg_ref, w1_ref, w3_ref, w2_ref, yg_ref):
    cb = pl.program_id(1)

    @pl.when(cb * _B < cnt_ref[0, pl.program_id(0)])
    def _():
        xgb = xg_ref[...].astype(jnp.bfloat16)
        g = _dot_t(xgb, w1_ref[0].astype(jnp.bfloat16))
        u = _dot_t(xgb, w3_ref[0].astype(jnp.bfloat16))
        h = (g * jax.nn.sigmoid(g) * u).astype(jnp.bfloat16)
        yg_ref[...] = _dot_t(h, w2_ref[0].astype(jnp.bfloat16))


def _shared_body(xb_ref, sw1_ref, sw3_ref, sw2_ref, o_ref):
    xb = xb_ref[...]
    g = _dot_t(xb, sw1_ref[...].astype(jnp.bfloat16))
    u = _dot_t(xb, sw3_ref[...].astype(jnp.bfloat16))
    h = (g * jax.nn.sigmoid(g) * u).astype(jnp.bfloat16)
    o_ref[...] = _dot_t(h, sw2_ref[...].astype(jnp.bfloat16))


def _dispatch_body(x_hbm, idx_hbm, xg_hbm, i_v, rows_v, semi, semr):
    c = jax.lax.axis_index("c")
    s = jax.lax.axis_index("s")
    wid = c * _NS + s
    j0 = wid * (2 * _T // _NW)
    r = j0 // _T
    t0 = j0 % _T

    def load(cc):
        tb = t0 + cc * 32
        ldi = pltpu.async_copy(
            idx_hbm.at[r, pl.ds(tb, 32)], i_v.at[cc % 2], semi)
        ldr = pltpu.async_copy(
            x_hbm.at[pl.ds(tb, 32)], rows_v.at[cc % 2], semr)
        return ldi, ldr

    pend = [load(0), load(1)]
    for cc in range(4):
        ldi, ldr = pend[cc % 2]
        ldi.wait()
        ldr.wait()
        pltpu.sync_copy(rows_v.at[cc % 2], xg_hbm.at[i_v.at[cc % 2]])
        if cc + 2 < 4:
            pend[cc % 2] = load(cc + 2)


def _combine_body(yg_hbm, idx_hbm, wexp_hbm, sh_hbm, out_hbm,
                  i_v, w_v, sh_v, y0_v, y1_v, sems):
    c = jax.lax.axis_index("c")
    s = jax.lax.axis_index("s")
    wid = c * _NS + s
    t0 = wid * (_T // _NW)
    nch = _T // _NW // 16

    for cc in range(nch):
        b = cc % 2
        tb = t0 + cc * 16
        pltpu.sync_copy(idx_hbm.at[0, pl.ds(tb, 16)], i_v.at[b, 0])
        pltpu.sync_copy(idx_hbm.at[1, pl.ds(tb, 16)], i_v.at[b, 1])
        pltpu.sync_copy(wexp_hbm.at[0, pl.ds(tb, 16)], w_v.at[b, 0])
        pltpu.sync_copy(wexp_hbm.at[1, pl.ds(tb, 16)], w_v.at[b, 1])
        pltpu.sync_copy(sh_hbm.at[pl.ds(tb, 16)], sh_v.at[b])
        g0 = pltpu.async_copy(yg_hbm.at[i_v.at[b, 0]], y0_v.at[b],
                              sems.at[0])
        g1 = pltpu.async_copy(yg_hbm.at[i_v.at[b, 1]], y1_v.at[b],
                              sems.at[1])
        g0.wait()
        g1.wait()

        def tok_body(i, carry):
            w0s = w_v[b, 0, i]
            w1s = w_v[b, 1, i]

            def col_body(k, carry2):
                sl = pl.ds(k * 16, 16)
                sh_v[b, i, sl] = (sh_v[b, i, sl] + w0s * y0_v[b, i, sl]
                                  + w1s * y1_v[b, i, sl])
                return carry2

            return jax.lax.fori_loop(0, _D // 16, col_body, carry, unroll=4)

        jax.lax.fori_loop(0, 16, tok_body, 0)
        pltpu.sync_copy(sh_v.at[b], out_hbm.at[pl.ds(tb, 16)])


def _sc_mesh():
    return plsc.VectorSubcoreMesh(
        core_axis_name="c", subcore_axis_name="s",
        num_cores=_NC, num_subcores=_NS)


def _sc_dispatch(x, idx):
    return pl.kernel(
        _dispatch_body,
        out_type=jax.ShapeDtypeStruct((_XG_ROWS, _D), jnp.float32),
        mesh=_sc_mesh(),
        scratch_types=[pltpu.VMEM((2, 32), jnp.int32),
                       pltpu.VMEM((2, 32, _D), jnp.float32),
                       pltpu.SemaphoreType.DMA,
                       pltpu.SemaphoreType.DMA],
    )(x, idx)


def _sc_combine(yg, idx, wexp, shared):
    return pl.kernel(
        _combine_body,
        out_type=jax.ShapeDtypeStruct((_T, _D), jnp.float32),
        mesh=_sc_mesh(),
        scratch_types=[pltpu.VMEM((2, 2, 16), jnp.int32),
                       pltpu.VMEM((2, 2, 16, 16), jnp.float32),
                       pltpu.VMEM((2, 16, _D), jnp.float32),
                       pltpu.VMEM((2, 16, _D), jnp.float32),
                       pltpu.VMEM((2, 16, _D), jnp.float32),
                       pltpu.SemaphoreType.DMA((4,))],
    )(yg, idx, wexp, shared)


def kernel(hidden_states, gate_w, expert_bias, w1, w3, w2, sw1, sw3, sw2):
    x = hidden_states.reshape(_T, _D)
    eb = expert_bias.reshape(1, _E)
    meta, cnts = pl.pallas_call(
        _router_body,
        out_shape=(jax.ShapeDtypeStruct((_T, 128), jnp.float32),
                   jax.ShapeDtypeStruct((1, _E), jnp.float32)),
    )(x, gate_w, eb)
    idx = jnp.transpose(meta[:, 0:2]).astype(jnp.int32)
    wexp = jnp.broadcast_to(
        jnp.transpose(meta[:, 2:4])[:, :, None], (2, _T, 16))
    counts = cnts.astype(jnp.int32)
    xb = x.astype(jnp.bfloat16)

    xg = _sc_dispatch(x, idx)

    shared = pl.pallas_call(
        _shared_body,
        out_shape=jax.ShapeDtypeStruct((_T, _D), jnp.float32),
    )(xb, sw1, sw3, sw2)

    yg = pl.pallas_call(
        _group_body,
        grid=(_E, _C // _B),
        in_specs=[
            pl.BlockSpec(memory_space=pltpu.SMEM),
            pl.BlockSpec((_B, _D), lambda e, cb: (e * (_C // _B) + cb, 0)),
            pl.BlockSpec((1, _DFF, _D), lambda e, cb: (e, 0, 0)),
            pl.BlockSpec((1, _DFF, _D), lambda e, cb: (e, 0, 0)),
            pl.BlockSpec((1, _D, _DFF), lambda e, cb: (e, 0, 0)),
        ],
        out_specs=pl.BlockSpec((_B, _D), lambda e, cb: (e * (_C // _B) + cb, 0)),
        out_shape=jax.ShapeDtypeStruct((_XG_ROWS, _D), jnp.float32),
    )(counts, xg, w1, w3, w2)

    out = _sc_combine(yg, idx, wexp, shared)
    return out


# speedup vs baseline: 2.1255x; 1.0466x over previous
import functools

import jax
import jax.numpy as jnp
from jax.experimental import pallas as pl
from jax.experimental.pallas import tpu as pltpu
from jax.experimental.pallas import tpu_sc as plsc

_T, _D, _E, _TOPK, _NG, _TG, _DFF, _DFFS = 2048, 1024, 16, 2, 4, 2, 512, 512
_GS = _E // _NG
_ROUTE_SCALE = 2.5
_C = 512
_B = 128
_S = _E * _C
_XG_ROWS = _S + _B
_NC, _NS = 2, 16
_NW = _NC * _NS


def _router_body(x_ref, gw_ref, eb_ref, meta_ref, cnt_ref):
    x = x_ref[...]
    logits = jax.lax.dot_general(
        x, gw_ref[...], (((1,), (1,)), ((), ())),
        preferred_element_type=jnp.float32)
    scores = jax.nn.sigmoid(logits)
    sfc = scores + eb_ref[...]
    gs_cols = []
    for g in range(_NG):
        c = [sfc[:, g * _GS + i:g * _GS + i + 1] for i in range(_GS)]
        best = None
        for i in range(_GS):
            for j in range(i + 1, _GS):
                s = c[i] + c[j]
                best = s if best is None else jnp.maximum(best, s)
        gs_cols.append(best)
    gs = jnp.concatenate(gs_cols, axis=1)
    gidx = jax.lax.broadcasted_iota(jnp.int32, (_T, _NG), 1)
    grank = jnp.zeros((_T, _NG), jnp.float32)
    for j in range(_NG):
        gj = gs[:, j:j + 1]
        grank += jnp.where((gj > gs) | ((gj == gs) & (j < gidx)), 1.0, 0.0)
    gsel = (grank < _TG).astype(jnp.float32)
    emask = jnp.concatenate(
        [gsel[:, e // _GS:e // _GS + 1] for e in range(_E)], axis=1)
    tmp = sfc * emask
    eidx = jax.lax.broadcasted_iota(jnp.int32, (_T, _E), 1)
    erank = jnp.zeros((_T, _E), jnp.float32)
    for j in range(_E):
        vj = tmp[:, j:j + 1]
        erank += jnp.where((vj > tmp) | ((vj == tmp) & (j < eidx)), 1.0, 0.0)
    sel = jnp.where(erank < _TOPK, 1.0, 0.0)
    w = scores * sel
    denom = jnp.sum(w, axis=1, keepdims=True) + 1e-20
    wfull = w * (_ROUTE_SCALE / denom)
    selb = sel.astype(jnp.bfloat16)
    riota = jax.lax.broadcasted_iota(jnp.int32, (_T, 1), 0)
    ciota = jax.lax.broadcasted_iota(jnp.int32, (1, _T), 1)
    ltri = (riota >= ciota).astype(jnp.bfloat16)
    pos = jax.lax.dot_general(
        ltri, selb, (((1,), (0,)), ((), ())),
        preferred_element_type=jnp.float32)
    cnt_ref[...] = pos[_T - 1:_T, :]
    eidx_f = eidx.astype(jnp.float32)
    slot = eidx_f * _C + (pos - 1.0)
    slot = jnp.where(pos - 1.0 < _C, slot, float(_S))
    r16 = jax.lax.broadcasted_iota(jnp.int32, (_E, _E), 0)
    c16 = jax.lax.broadcasted_iota(jnp.int32, (_E, _E), 1)
    ltri16 = (r16 <= c16).astype(jnp.bfloat16)
    cl = jax.lax.dot_general(
        selb, ltri16, (((1,), (0,)), ((), ())),
        preferred_element_type=jnp.float32)
    low = sel * jnp.where(cl == 1.0, 1.0, 0.0)
    high = sel * jnp.where(cl == 2.0, 1.0, 0.0)
    dst0 = jnp.sum(slot * low, axis=1, keepdims=True)
    dst1 = jnp.sum(slot * high, axis=1, keepdims=True)
    w0 = jnp.sum(wfull * low, axis=1, keepdims=True)
    w1 = jnp.sum(wfull * high, axis=1, keepdims=True)
    meta_ref[:, 0:4] = jnp.concatenate([dst0, dst1, w0, w1], axis=1)


def _dot_t(a, b):
    return jax.lax.dot_general(
        a, b, (((1,), (1,)), ((), ())), preferred_element_type=jnp.float32)


def _group_body(cnt_ref, xg_ref, w1_ref, w3_ref, w2_ref, yg_ref):
    cb = pl.program_id(1)

    @pl.when(cb * _B < cnt_ref[0, pl.program_id(0)])
    def _():
        xgb = xg_ref[...].astype(jnp.bfloat16)
        g = _dot_t(xgb, w1_ref[0].astype(jnp.bfloat16))
        u = _dot_t(xgb, w3_ref[0].astype(jnp.bfloat16))
        h = (g * jax.nn.sigmoid(g) * u).astype(jnp.bfloat16)
        yg_ref[...] = _dot_t(h, w2_ref[0].astype(jnp.bfloat16))


def _shared_body(xb_ref, sw1_ref, sw3_ref, sw2_ref, o_ref):
    xb = xb_ref[...]
    g = _dot_t(xb, sw1_ref[...].astype(jnp.bfloat16))
    u = _dot_t(xb, sw3_ref[...].astype(jnp.bfloat16))
    h = (g * jax.nn.sigmoid(g) * u).astype(jnp.bfloat16)
    o_ref[...] = _dot_t(h, sw2_ref[...].astype(jnp.bfloat16))


def _dispatch_body(x_hbm, idx_hbm, xg_hbm, i_v, rows_v, semi, semr):
    c = jax.lax.axis_index("c")
    s = jax.lax.axis_index("s")
    wid = c * _NS + s
    j0 = wid * (2 * _T // _NW)
    r = j0 // _T
    t0 = j0 % _T

    def load(cc):
        tb = t0 + cc * 32
        ldi = pltpu.async_copy(
            idx_hbm.at[r, pl.ds(tb, 32)], i_v.at[cc % 2], semi)
        ldr = pltpu.async_copy(
            x_hbm.at[pl.ds(tb, 32)], rows_v.at[cc % 2], semr)
        return ldi, ldr

    pend = [load(0), load(1)]
    for cc in range(4):
        ldi, ldr = pend[cc % 2]
        ldi.wait()
        ldr.wait()
        pltpu.sync_copy(rows_v.at[cc % 2], xg_hbm.at[i_v.at[cc % 2]])
        if cc + 2 < 4:
            pend[cc % 2] = load(cc + 2)


def _combine_body(yg_hbm, idx_hbm, wexp_hbm, sh_hbm, out_hbm,
                  i_v, w_v, sh_v, y0_v, y1_v, sems):
    c = jax.lax.axis_index("c")
    s = jax.lax.axis_index("s")
    wid = c * _NS + s
    t0 = wid * (_T // _NW)

    pltpu.sync_copy(idx_hbm.at[0, pl.ds(t0, 64)], i_v.at[0])
    pltpu.sync_copy(idx_hbm.at[1, pl.ds(t0, 64)], i_v.at[1])
    pltpu.sync_copy(wexp_hbm.at[0, pl.ds(t0, 64)], w_v.at[0])
    pltpu.sync_copy(wexp_hbm.at[1, pl.ds(t0, 64)], w_v.at[1])
    for h in range(2):
        tb = t0 + h * 32
        g0 = pltpu.async_copy(yg_hbm.at[i_v.at[0, pl.ds(h * 32, 32)]],
                              y0_v, sems.at[0])
        g1 = pltpu.async_copy(yg_hbm.at[i_v.at[1, pl.ds(h * 32, 32)]],
                              y1_v, sems.at[1])
        pltpu.sync_copy(sh_hbm.at[pl.ds(tb, 32)], sh_v)
        g0.wait()
        g1.wait()

        def tok_body(i, carry):
            w0s = w_v[0, h * 32 + i]
            w1s = w_v[1, h * 32 + i]

            def col_body(k, carry2):
                sl = pl.ds(k * 16, 16)
                sh_v[i, sl] = (sh_v[i, sl] + w0s * y0_v[i, sl]
                               + w1s * y1_v[i, sl])
                return carry2

            return jax.lax.fori_loop(0, _D // 16, col_body, carry, unroll=4)

        jax.lax.fori_loop(0, 32, tok_body, 0)
        pltpu.sync_copy(sh_v, out_hbm.at[pl.ds(tb, 32)])


def _sc_mesh():
    return plsc.VectorSubcoreMesh(
        core_axis_name="c", subcore_axis_name="s",
        num_cores=_NC, num_subcores=_NS)


def _sc_dispatch(x, idx):
    return pl.kernel(
        _dispatch_body,
        out_type=jax.ShapeDtypeStruct((_XG_ROWS, _D), jnp.float32),
        mesh=_sc_mesh(),
        scratch_types=[pltpu.VMEM((2, 32), jnp.int32),
                       pltpu.VMEM((2, 32, _D), jnp.float32),
                       pltpu.SemaphoreType.DMA,
                       pltpu.SemaphoreType.DMA],
    )(x, idx)


def _sc_combine(yg, idx, wexp, shared):
    return pl.kernel(
        _combine_body,
        out_type=jax.ShapeDtypeStruct((_T, _D), jnp.float32),
        mesh=_sc_mesh(),
        scratch_types=[pltpu.VMEM((2, 64), jnp.int32),
                       pltpu.VMEM((2, 64, 16), jnp.float32),
                       pltpu.VMEM((32, _D), jnp.float32),
                       pltpu.VMEM((32, _D), jnp.float32),
                       pltpu.VMEM((32, _D), jnp.float32),
                       pltpu.SemaphoreType.DMA((2,))],
    )(yg, idx, wexp, shared)


def kernel(hidden_states, gate_w, expert_bias, w1, w3, w2, sw1, sw3, sw2):
    x = hidden_states.reshape(_T, _D)
    eb = expert_bias.reshape(1, _E)
    meta, cnts = pl.pallas_call(
        _router_body,
        out_shape=(jax.ShapeDtypeStruct((_T, 128), jnp.float32),
                   jax.ShapeDtypeStruct((1, _E), jnp.float32)),
    )(x, gate_w, eb)
    idx = jnp.transpose(meta[:, 0:2]).astype(jnp.int32)
    wexp = jnp.broadcast_to(
        jnp.transpose(meta[:, 2:4])[:, :, None], (2, _T, 16))
    counts = cnts.astype(jnp.int32)
    xb = x.astype(jnp.bfloat16)

    xg = _sc_dispatch(x, idx)

    shared = pl.pallas_call(
        _shared_body,
        out_shape=jax.ShapeDtypeStruct((_T, _D), jnp.float32),
    )(xb, sw1, sw3, sw2)

    yg = pl.pallas_call(
        _group_body,
        grid=(_E, _C // _B),
        in_specs=[
            pl.BlockSpec(memory_space=pltpu.SMEM),
            pl.BlockSpec((_B, _D), lambda e, cb: (e * (_C // _B) + cb, 0)),
            pl.BlockSpec((1, _DFF, _D), lambda e, cb: (e, 0, 0)),
            pl.BlockSpec((1, _DFF, _D), lambda e, cb: (e, 0, 0)),
            pl.BlockSpec((1, _D, _DFF), lambda e, cb: (e, 0, 0)),
        ],
        out_specs=pl.BlockSpec((_B, _D), lambda e, cb: (e * (_C // _B) + cb, 0)),
        out_shape=jax.ShapeDtypeStruct((_XG_ROWS, _D), jnp.float32),
    )(counts, xg, w1, w3, w2)

    out = _sc_combine(yg, idx, wexp, shared)
    return out


# speedup vs baseline: 2.1349x; 1.0044x over previous
import functools

import jax
import jax.numpy as jnp
from jax.experimental import pallas as pl
from jax.experimental.pallas import tpu as pltpu
from jax.experimental.pallas import tpu_sc as plsc

_T, _D, _E, _TOPK, _NG, _TG, _DFF, _DFFS = 2048, 1024, 16, 2, 4, 2, 512, 512
_GS = _E // _NG
_ROUTE_SCALE = 2.5
_C = 512
_B = 128
_S = _E * _C
_XG_ROWS = _S + _B
_NC, _NS = 2, 16
_NW = _NC * _NS


def _router_body(x_ref, gw_ref, eb_ref, meta_ref, cnt_ref):
    x = x_ref[...]
    logits = jax.lax.dot_general(
        x, gw_ref[...], (((1,), (1,)), ((), ())),
        preferred_element_type=jnp.float32)
    scores = jax.nn.sigmoid(logits)
    sfc = scores + eb_ref[...]
    gs_cols = []
    for g in range(_NG):
        c = [sfc[:, g * _GS + i:g * _GS + i + 1] for i in range(_GS)]
        best = None
        for i in range(_GS):
            for j in range(i + 1, _GS):
                s = c[i] + c[j]
                best = s if best is None else jnp.maximum(best, s)
        gs_cols.append(best)
    gs = jnp.concatenate(gs_cols, axis=1)
    gidx = jax.lax.broadcasted_iota(jnp.int32, (_T, _NG), 1)
    grank = jnp.zeros((_T, _NG), jnp.float32)
    for j in range(_NG):
        gj = gs[:, j:j + 1]
        grank += jnp.where((gj > gs) | ((gj == gs) & (j < gidx)), 1.0, 0.0)
    gsel = (grank < _TG).astype(jnp.float32)
    emask = jnp.concatenate(
        [gsel[:, e // _GS:e // _GS + 1] for e in range(_E)], axis=1)
    tmp = sfc * emask
    eidx = jax.lax.broadcasted_iota(jnp.int32, (_T, _E), 1)
    erank = jnp.zeros((_T, _E), jnp.float32)
    for j in range(_E):
        vj = tmp[:, j:j + 1]
        erank += jnp.where((vj > tmp) | ((vj == tmp) & (j < eidx)), 1.0, 0.0)
    sel = jnp.where(erank < _TOPK, 1.0, 0.0)
    w = scores * sel
    denom = jnp.sum(w, axis=1, keepdims=True) + 1e-20
    wfull = w * (_ROUTE_SCALE / denom)
    selb = sel.astype(jnp.bfloat16)
    nb, bb = 16, _T // 16
    rb = jax.lax.broadcasted_iota(jnp.int32, (bb, 1), 0)
    cb2 = jax.lax.broadcasted_iota(jnp.int32, (1, bb), 1)
    l128 = (rb >= cb2).astype(jnp.bfloat16)
    pos_blocks = []
    tot_rows = []
    for b in range(nb):
        sb = selb[b * bb:(b + 1) * bb, :]
        pb = jax.lax.dot_general(
            l128, sb, (((1,), (0,)), ((), ())),
            preferred_element_type=jnp.float32)
        pos_blocks.append(pb)
        tot_rows.append(pb[bb - 1:bb, :])
    tot = jnp.concatenate(tot_rows, axis=0).astype(jnp.bfloat16)
    rnb = jax.lax.broadcasted_iota(jnp.int32, (nb, nb), 0)
    cnb = jax.lax.broadcasted_iota(jnp.int32, (nb, nb), 1)
    lstrict = (rnb > cnb).astype(jnp.bfloat16)
    off = jax.lax.dot_general(
        lstrict, tot, (((1,), (0,)), ((), ())),
        preferred_element_type=jnp.float32)
    pos = jnp.concatenate(
        [pos_blocks[b] + off[b:b + 1, :] for b in range(nb)],
        axis=0)
    cnt_ref[...] = pos[_T - 1:_T, :]
    eidx_f = eidx.astype(jnp.float32)
    slot = eidx_f * _C + (pos - 1.0)
    slot = jnp.where(pos - 1.0 < _C, slot, float(_S))
    r16 = jax.lax.broadcasted_iota(jnp.int32, (_E, _E), 0)
    c16 = jax.lax.broadcasted_iota(jnp.int32, (_E, _E), 1)
    ltri16 = (r16 <= c16).astype(jnp.bfloat16)
    cl = jax.lax.dot_general(
        selb, ltri16, (((1,), (0,)), ((), ())),
        preferred_element_type=jnp.float32)
    low = sel * jnp.where(cl == 1.0, 1.0, 0.0)
    high = sel * jnp.where(cl == 2.0, 1.0, 0.0)
    dst0 = jnp.sum(slot * low, axis=1, keepdims=True)
    dst1 = jnp.sum(slot * high, axis=1, keepdims=True)
    w0 = jnp.sum(wfull * low, axis=1, keepdims=True)
    w1 = jnp.sum(wfull * high, axis=1, keepdims=True)
    meta_ref[:, 0:4] = jnp.concatenate([dst0, dst1, w0, w1], axis=1)


def _dot_t(a, b):
    return jax.lax.dot_general(
        a, b, (((1,), (1,)), ((), ())), preferred_element_type=jnp.float32)


def _group_body(cnt_ref, xg_ref, w1_ref, w3_ref, w2_ref, yg_ref):
    cb = pl.program_id(1)

    @pl.when(cb * _B < cnt_ref[0, pl.program_id(0)])
    def _():
        xgb = xg_ref[...].astype(jnp.bfloat16)
        g = _dot_t(xgb, w1_ref[0].astype(jnp.bfloat16))
        u = _dot_t(xgb, w3_ref[0].astype(jnp.bfloat16))
        h = (g * jax.nn.sigmoid(g) * u).astype(jnp.bfloat16)
        yg_ref[...] = _dot_t(h, w2_ref[0].astype(jnp.bfloat16))


def _shared_body(xb_ref, sw1_ref, sw3_ref, sw2_ref, o_ref):
    xb = xb_ref[...]
    g = _dot_t(xb, sw1_ref[...].astype(jnp.bfloat16))
    u = _dot_t(xb, sw3_ref[...].astype(jnp.bfloat16))
    h = (g * jax.nn.sigmoid(g) * u).astype(jnp.bfloat16)
    o_ref[...] = _dot_t(h, sw2_ref[...].astype(jnp.bfloat16))


def _dispatch_body(x_hbm, idx_hbm, xg_hbm, i_v, rows_v, semi, semr):
    c = jax.lax.axis_index("c")
    s = jax.lax.axis_index("s")
    wid = c * _NS + s
    j0 = wid * (2 * _T // _NW)
    r = j0 // _T
    t0 = j0 % _T

    def load(cc):
        tb = t0 + cc * 32
        ldi = pltpu.async_copy(
            idx_hbm.at[r, pl.ds(tb, 32)], i_v.at[cc % 2], semi)
        ldr = pltpu.async_copy(
            x_hbm.at[pl.ds(tb, 32)], rows_v.at[cc % 2], semr)
        return ldi, ldr

    pend = [load(0), load(1)]
    for cc in range(4):
        ldi, ldr = pend[cc % 2]
        ldi.wait()
        ldr.wait()
        pltpu.sync_copy(rows_v.at[cc % 2], xg_hbm.at[i_v.at[cc % 2]])
        if cc + 2 < 4:
            pend[cc % 2] = load(cc + 2)


def _combine_body(yg_hbm, idx_hbm, wexp_hbm, sh_hbm, out_hbm,
                  i_v, w_v, sh_v, y0_v, y1_v, sems):
    c = jax.lax.axis_index("c")
    s = jax.lax.axis_index("s")
    wid = c * _NS + s
    t0 = wid * (_T // _NW)

    pltpu.sync_copy(idx_hbm.at[0, pl.ds(t0, 64)], i_v.at[0])
    pltpu.sync_copy(idx_hbm.at[1, pl.ds(t0, 64)], i_v.at[1])
    pltpu.sync_copy(wexp_hbm.at[0, pl.ds(t0, 64)], w_v.at[0])
    pltpu.sync_copy(wexp_hbm.at[1, pl.ds(t0, 64)], w_v.at[1])
    for h in range(2):
        tb = t0 + h * 32
        g0 = pltpu.async_copy(yg_hbm.at[i_v.at[0, pl.ds(h * 32, 32)]],
                              y0_v, sems.at[0])
        g1 = pltpu.async_copy(yg_hbm.at[i_v.at[1, pl.ds(h * 32, 32)]],
                              y1_v, sems.at[1])
        pltpu.sync_copy(sh_hbm.at[pl.ds(tb, 32)], sh_v)
        g0.wait()
        g1.wait()

        def tok_body(i, carry):
            w0s = w_v[0, h * 32 + i]
            w1s = w_v[1, h * 32 + i]

            def col_body(k, carry2):
                sl = pl.ds(k * 16, 16)
                sh_v[i, sl] = (sh_v[i, sl] + w0s * y0_v[i, sl]
                               + w1s * y1_v[i, sl])
                return carry2

            return jax.lax.fori_loop(0, _D // 16, col_body, carry, unroll=4)

        jax.lax.fori_loop(0, 32, tok_body, 0)
        pltpu.sync_copy(sh_v, out_hbm.at[pl.ds(tb, 32)])


def _sc_mesh():
    return plsc.VectorSubcoreMesh(
        core_axis_name="c", subcore_axis_name="s",
        num_cores=_NC, num_subcores=_NS)


def _sc_dispatch(x, idx):
    return pl.kernel(
        _dispatch_body,
        out_type=jax.ShapeDtypeStruct((_XG_ROWS, _D), jnp.float32),
        mesh=_sc_mesh(),
        scratch_types=[pltpu.VMEM((2, 32), jnp.int32),
                       pltpu.VMEM((2, 32, _D), jnp.float32),
                       pltpu.SemaphoreType.DMA,
                       pltpu.SemaphoreType.DMA],
    )(x, idx)


def _sc_combine(yg, idx, wexp, shared):
    return pl.kernel(
        _combine_body,
        out_type=jax.ShapeDtypeStruct((_T, _D), jnp.float32),
        mesh=_sc_mesh(),
        scratch_types=[pltpu.VMEM((2, 64), jnp.int32),
                       pltpu.VMEM((2, 64, 16), jnp.float32),
                       pltpu.VMEM((32, _D), jnp.float32),
                       pltpu.VMEM((32, _D), jnp.float32),
                       pltpu.VMEM((32, _D), jnp.float32),
                       pltpu.SemaphoreType.DMA((2,))],
    )(yg, idx, wexp, shared)


def kernel(hidden_states, gate_w, expert_bias, w1, w3, w2, sw1, sw3, sw2):
    x = hidden_states.reshape(_T, _D)
    eb = expert_bias.reshape(1, _E)
    meta, cnts = pl.pallas_call(
        _router_body,
        out_shape=(jax.ShapeDtypeStruct((_T, 128), jnp.float32),
                   jax.ShapeDtypeStruct((1, _E), jnp.float32)),
    )(x, gate_w, eb)
    idx = jnp.transpose(meta[:, 0:2]).astype(jnp.int32)
    wexp = jnp.broadcast_to(
        jnp.transpose(meta[:, 2:4])[:, :, None], (2, _T, 16))
    counts = cnts.astype(jnp.int32)
    xb = x.astype(jnp.bfloat16)

    xg = _sc_dispatch(x, idx)

    shared = pl.pallas_call(
        _shared_body,
        out_shape=jax.ShapeDtypeStruct((_T, _D), jnp.float32),
    )(xb, sw1, sw3, sw2)

    yg = pl.pallas_call(
        _group_body,
        grid=(_E, _C // _B),
        in_specs=[
            pl.BlockSpec(memory_space=pltpu.SMEM),
            pl.BlockSpec((_B, _D), lambda e, cb: (e * (_C // _B) + cb, 0)),
            pl.BlockSpec((1, _DFF, _D), lambda e, cb: (e, 0, 0)),
            pl.BlockSpec((1, _DFF, _D), lambda e, cb: (e, 0, 0)),
            pl.BlockSpec((1, _D, _DFF), lambda e, cb: (e, 0, 0)),
        ],
        out_specs=pl.BlockSpec((_B, _D), lambda e, cb: (e * (_C // _B) + cb, 0)),
        out_shape=jax.ShapeDtypeStruct((_XG_ROWS, _D), jnp.float32),
    )(counts, xg, w1, w3, w2)

    out = _sc_combine(yg, idx, wexp, shared)
    return out


# speedup vs baseline: 2.1994x; 1.0302x over previous
import functools

import jax
import jax.numpy as jnp
from jax.experimental import pallas as pl
from jax.experimental.pallas import tpu as pltpu
from jax.experimental.pallas import tpu_sc as plsc

_T, _D, _E, _TOPK, _NG, _TG, _DFF, _DFFS = 2048, 1024, 16, 2, 4, 2, 512, 512
_GS = _E // _NG
_ROUTE_SCALE = 2.5
_C = 512
_B = 128
_S = _E * _C
_XG_ROWS = _S + _B
_NC, _NS = 2, 16
_NW = _NC * _NS


def _router_body(x_ref, gw_ref, eb_ref, sw1_ref, sw3_ref, sw2_ref,
                 meta_ref, cnt_ref, sh_ref):
    x = x_ref[...]
    xbv = x.astype(jnp.bfloat16)
    gsh = _dot_t(xbv, sw1_ref[...].astype(jnp.bfloat16))
    ush = _dot_t(xbv, sw3_ref[...].astype(jnp.bfloat16))
    hsh = (gsh * jax.nn.sigmoid(gsh) * ush).astype(jnp.bfloat16)
    sh_ref[...] = _dot_t(hsh, sw2_ref[...].astype(jnp.bfloat16))
    logits = jax.lax.dot_general(
        x, gw_ref[...], (((1,), (1,)), ((), ())),
        preferred_element_type=jnp.float32)
    scores = jax.nn.sigmoid(logits)
    sfc = scores + eb_ref[...]
    gs_cols = []
    for g in range(_NG):
        c = [sfc[:, g * _GS + i:g * _GS + i + 1] for i in range(_GS)]
        best = None
        for i in range(_GS):
            for j in range(i + 1, _GS):
                s = c[i] + c[j]
                best = s if best is None else jnp.maximum(best, s)
        gs_cols.append(best)
    gs = jnp.concatenate(gs_cols, axis=1)
    gidx = jax.lax.broadcasted_iota(jnp.int32, (_T, _NG), 1)
    grank = jnp.zeros((_T, _NG), jnp.float32)
    for j in range(_NG):
        gj = gs[:, j:j + 1]
        grank += jnp.where((gj > gs) | ((gj == gs) & (j < gidx)), 1.0, 0.0)
    gsel = (grank < _TG).astype(jnp.float32)
    emask = jnp.concatenate(
        [gsel[:, e // _GS:e // _GS + 1] for e in range(_E)], axis=1)
    tmp = sfc * emask
    eidx = jax.lax.broadcasted_iota(jnp.int32, (_T, _E), 1)
    erank = jnp.zeros((_T, _E), jnp.float32)
    for j in range(_E):
        vj = tmp[:, j:j + 1]
        erank += jnp.where((vj > tmp) | ((vj == tmp) & (j < eidx)), 1.0, 0.0)
    sel = jnp.where(erank < _TOPK, 1.0, 0.0)
    w = scores * sel
    denom = jnp.sum(w, axis=1, keepdims=True) + 1e-20
    wfull = w * (_ROUTE_SCALE / denom)
    selb = sel.astype(jnp.bfloat16)
    nb, bb = 16, _T // 16
    rb = jax.lax.broadcasted_iota(jnp.int32, (bb, 1), 0)
    cb2 = jax.lax.broadcasted_iota(jnp.int32, (1, bb), 1)
    l128 = (rb >= cb2).astype(jnp.bfloat16)
    pos_blocks = []
    tot_rows = []
    for b in range(nb):
        sb = selb[b * bb:(b + 1) * bb, :]
        pb = jax.lax.dot_general(
            l128, sb, (((1,), (0,)), ((), ())),
            preferred_element_type=jnp.float32)
        pos_blocks.append(pb)
        tot_rows.append(pb[bb - 1:bb, :])
    tot = jnp.concatenate(tot_rows, axis=0).astype(jnp.bfloat16)
    rnb = jax.lax.broadcasted_iota(jnp.int32, (nb, nb), 0)
    cnb = jax.lax.broadcasted_iota(jnp.int32, (nb, nb), 1)
    lstrict = (rnb > cnb).astype(jnp.bfloat16)
    off = jax.lax.dot_general(
        lstrict, tot, (((1,), (0,)), ((), ())),
        preferred_element_type=jnp.float32)
    pos = jnp.concatenate(
        [pos_blocks[b] + off[b:b + 1, :] for b in range(nb)],
        axis=0)
    cnt_ref[...] = pos[_T - 1:_T, :]
    eidx_f = eidx.astype(jnp.float32)
    slot = eidx_f * _C + (pos - 1.0)
    slot = jnp.where(pos - 1.0 < _C, slot, float(_S))
    r16 = jax.lax.broadcasted_iota(jnp.int32, (_E, _E), 0)
    c16 = jax.lax.broadcasted_iota(jnp.int32, (_E, _E), 1)
    ltri16 = (r16 <= c16).astype(jnp.bfloat16)
    cl = jax.lax.dot_general(
        selb, ltri16, (((1,), (0,)), ((), ())),
        preferred_element_type=jnp.float32)
    low = sel * jnp.where(cl == 1.0, 1.0, 0.0)
    high = sel * jnp.where(cl == 2.0, 1.0, 0.0)
    dst0 = jnp.sum(slot * low, axis=1, keepdims=True)
    dst1 = jnp.sum(slot * high, axis=1, keepdims=True)
    w0 = jnp.sum(wfull * low, axis=1, keepdims=True)
    w1 = jnp.sum(wfull * high, axis=1, keepdims=True)
    meta_ref[:, 0:4] = jnp.concatenate([dst0, dst1, w0, w1], axis=1)


def _dot_t(a, b):
    return jax.lax.dot_general(
        a, b, (((1,), (1,)), ((), ())), preferred_element_type=jnp.float32)


def _group_body(cnt_ref, xg_ref, w1_ref, w3_ref, w2_ref, yg_ref):
    cb = pl.program_id(1)

    @pl.when(cb * _B < cnt_ref[0, pl.program_id(0)])
    def _():
        xgb = xg_ref[...].astype(jnp.bfloat16)
        g = _dot_t(xgb, w1_ref[0].astype(jnp.bfloat16))
        u = _dot_t(xgb, w3_ref[0].astype(jnp.bfloat16))
        h = (g * jax.nn.sigmoid(g) * u).astype(jnp.bfloat16)
        yg_ref[...] = _dot_t(h, w2_ref[0].astype(jnp.bfloat16))


def _dispatch_body(x_hbm, idx_hbm, xg_hbm, i_v, rows_v, semi, semr):
    c = jax.lax.axis_index("c")
    s = jax.lax.axis_index("s")
    wid = c * _NS + s
    j0 = wid * (2 * _T // _NW)
    r = j0 // _T
    t0 = j0 % _T

    def load(cc):
        tb = t0 + cc * 32
        ldi = pltpu.async_copy(
            idx_hbm.at[r, pl.ds(tb, 32)], i_v.at[cc % 2], semi)
        ldr = pltpu.async_copy(
            x_hbm.at[pl.ds(tb, 32)], rows_v.at[cc % 2], semr)
        return ldi, ldr

    pend = [load(0), load(1)]
    for cc in range(4):
        ldi, ldr = pend[cc % 2]
        ldi.wait()
        ldr.wait()
        pltpu.sync_copy(rows_v.at[cc % 2], xg_hbm.at[i_v.at[cc % 2]])
        if cc + 2 < 4:
            pend[cc % 2] = load(cc + 2)


def _combine_body(yg_hbm, idx_hbm, wexp_hbm, sh_hbm, out_hbm,
                  i_v, w_v, sh_v, y0_v, y1_v, sems):
    c = jax.lax.axis_index("c")
    s = jax.lax.axis_index("s")
    wid = c * _NS + s
    t0 = wid * (_T // _NW)

    pltpu.sync_copy(idx_hbm.at[0, pl.ds(t0, 64)], i_v.at[0])
    pltpu.sync_copy(idx_hbm.at[1, pl.ds(t0, 64)], i_v.at[1])
    pltpu.sync_copy(wexp_hbm.at[0, pl.ds(t0, 64)], w_v.at[0])
    pltpu.sync_copy(wexp_hbm.at[1, pl.ds(t0, 64)], w_v.at[1])
    for h in range(2):
        tb = t0 + h * 32
        g0 = pltpu.async_copy(yg_hbm.at[i_v.at[0, pl.ds(h * 32, 32)]],
                              y0_v, sems.at[0])
        g1 = pltpu.async_copy(yg_hbm.at[i_v.at[1, pl.ds(h * 32, 32)]],
                              y1_v, sems.at[1])
        pltpu.sync_copy(sh_hbm.at[pl.ds(tb, 32)], sh_v)
        g0.wait()
        g1.wait()

        def tok_body(i, carry):
            w0s = w_v[0, h * 32 + i]
            w1s = w_v[1, h * 32 + i]

            def col_body(k, carry2):
                sl = pl.ds(k * 16, 16)
                sh_v[i, sl] = (sh_v[i, sl] + w0s * y0_v[i, sl]
                               + w1s * y1_v[i, sl])
                return carry2

            return jax.lax.fori_loop(0, _D // 16, col_body, carry, unroll=4)

        jax.lax.fori_loop(0, 32, tok_body, 0)
        pltpu.sync_copy(sh_v, out_hbm.at[pl.ds(tb, 32)])


def _sc_mesh():
    return plsc.VectorSubcoreMesh(
        core_axis_name="c", subcore_axis_name="s",
        num_cores=_NC, num_subcores=_NS)


def _sc_dispatch(x, idx):
    return pl.kernel(
        _dispatch_body,
        out_type=jax.ShapeDtypeStruct((_XG_ROWS, _D), jnp.float32),
        mesh=_sc_mesh(),
        scratch_types=[pltpu.VMEM((2, 32), jnp.int32),
                       pltpu.VMEM((2, 32, _D), jnp.float32),
                       pltpu.SemaphoreType.DMA,
                       pltpu.SemaphoreType.DMA],
    )(x, idx)


def _sc_combine(yg, idx, wexp, shared):
    return pl.kernel(
        _combine_body,
        out_type=jax.ShapeDtypeStruct((_T, _D), jnp.float32),
        mesh=_sc_mesh(),
        scratch_types=[pltpu.VMEM((2, 64), jnp.int32),
                       pltpu.VMEM((2, 64, 16), jnp.float32),
                       pltpu.VMEM((32, _D), jnp.float32),
                       pltpu.VMEM((32, _D), jnp.float32),
                       pltpu.VMEM((32, _D), jnp.float32),
                       pltpu.SemaphoreType.DMA((2,))],
    )(yg, idx, wexp, shared)


def kernel(hidden_states, gate_w, expert_bias, w1, w3, w2, sw1, sw3, sw2):
    x = hidden_states.reshape(_T, _D)
    eb = expert_bias.reshape(1, _E)
    meta, cnts, shared = pl.pallas_call(
        _router_body,
        out_shape=(jax.ShapeDtypeStruct((_T, 128), jnp.float32),
                   jax.ShapeDtypeStruct((1, _E), jnp.float32),
                   jax.ShapeDtypeStruct((_T, _D), jnp.float32)),
    )(x, gate_w, eb, sw1, sw3, sw2)
    idx = jnp.transpose(meta[:, 0:2]).astype(jnp.int32)
    wexp = jnp.broadcast_to(
        jnp.transpose(meta[:, 2:4])[:, :, None], (2, _T, 16))
    counts = cnts.astype(jnp.int32)

    xg = _sc_dispatch(x, idx)

    yg = pl.pallas_call(
        _group_body,
        grid=(_E, _C // _B),
        in_specs=[
            pl.BlockSpec(memory_space=pltpu.SMEM),
            pl.BlockSpec((_B, _D), lambda e, cb: (e * (_C // _B) + cb, 0)),
            pl.BlockSpec((1, _DFF, _D), lambda e, cb: (e, 0, 0)),
            pl.BlockSpec((1, _DFF, _D), lambda e, cb: (e, 0, 0)),
            pl.BlockSpec((1, _D, _DFF), lambda e, cb: (e, 0, 0)),
        ],
        out_specs=pl.BlockSpec((_B, _D), lambda e, cb: (e * (_C // _B) + cb, 0)),
        out_shape=jax.ShapeDtypeStruct((_XG_ROWS, _D), jnp.float32),
    )(counts, xg, w1, w3, w2)

    out = _sc_combine(yg, idx, wexp, shared)
    return out


# speedup vs baseline: 2.6248x; 1.1934x over previous
import functools

import jax
import jax.numpy as jnp
from jax.experimental import pallas as pl
from jax.experimental.pallas import tpu as pltpu
from jax.experimental.pallas import tpu_sc as plsc

_T, _D, _E, _TOPK, _NG, _TG, _DFF, _DFFS = 2048, 1024, 16, 2, 4, 2, 512, 512
_GS = _E // _NG
_ROUTE_SCALE = 2.5
_C = 512
_B = 256
_S = _E * _C
_XG_ROWS = _S + _B
_NC, _NS = 2, 16
_NW = _NC * _NS


def _router_body(x_ref, gw_ref, eb_ref, sw1_ref, sw3_ref, sw2_ref,
                 meta_ref, cnt_ref, sh_ref):
    x = x_ref[...]
    xbv = x.astype(jnp.bfloat16)
    gsh = _dot_t(xbv, sw1_ref[...].astype(jnp.bfloat16))
    ush = _dot_t(xbv, sw3_ref[...].astype(jnp.bfloat16))
    hsh = (gsh * jax.nn.sigmoid(gsh) * ush).astype(jnp.bfloat16)
    sh_ref[...] = _dot_t(hsh, sw2_ref[...].astype(jnp.bfloat16))
    logits = jax.lax.dot_general(
        x, gw_ref[...], (((1,), (1,)), ((), ())),
        preferred_element_type=jnp.float32)
    scores = jax.nn.sigmoid(logits)
    sfc = scores + eb_ref[...]
    gs_cols = []
    for g in range(_NG):
        c = [sfc[:, g * _GS + i:g * _GS + i + 1] for i in range(_GS)]
        best = None
        for i in range(_GS):
            for j in range(i + 1, _GS):
                s = c[i] + c[j]
                best = s if best is None else jnp.maximum(best, s)
        gs_cols.append(best)
    gs = jnp.concatenate(gs_cols, axis=1)
    gidx = jax.lax.broadcasted_iota(jnp.int32, (_T, _NG), 1)
    grank = jnp.zeros((_T, _NG), jnp.float32)
    for j in range(_NG):
        gj = gs[:, j:j + 1]
        grank += jnp.where((gj > gs) | ((gj == gs) & (j < gidx)), 1.0, 0.0)
    gsel = (grank < _TG).astype(jnp.float32)
    emask = jnp.concatenate(
        [gsel[:, e // _GS:e // _GS + 1] for e in range(_E)], axis=1)
    tmp = sfc * emask
    eidx = jax.lax.broadcasted_iota(jnp.int32, (_T, _E), 1)
    erank = jnp.zeros((_T, _E), jnp.float32)
    for j in range(_E):
        vj = tmp[:, j:j + 1]
        erank += jnp.where((vj > tmp) | ((vj == tmp) & (j < eidx)), 1.0, 0.0)
    sel = jnp.where(erank < _TOPK, 1.0, 0.0)
    w = scores * sel
    denom = jnp.sum(w, axis=1, keepdims=True) + 1e-20
    wfull = w * (_ROUTE_SCALE / denom)
    selb = sel.astype(jnp.bfloat16)
    nb, bb = 16, _T // 16
    rb = jax.lax.broadcasted_iota(jnp.int32, (bb, 1), 0)
    cb2 = jax.lax.broadcasted_iota(jnp.int32, (1, bb), 1)
    l128 = (rb >= cb2).astype(jnp.bfloat16)
    pos_blocks = []
    tot_rows = []
    for b in range(nb):
        sb = selb[b * bb:(b + 1) * bb, :]
        pb = jax.lax.dot_general(
            l128, sb, (((1,), (0,)), ((), ())),
            preferred_element_type=jnp.float32)
        pos_blocks.append(pb)
        tot_rows.append(pb[bb - 1:bb, :])
    tot = jnp.concatenate(tot_rows, axis=0).astype(jnp.bfloat16)
    rnb = jax.lax.broadcasted_iota(jnp.int32, (nb, nb), 0)
    cnb = jax.lax.broadcasted_iota(jnp.int32, (nb, nb), 1)
    lstrict = (rnb > cnb).astype(jnp.bfloat16)
    off = jax.lax.dot_general(
        lstrict, tot, (((1,), (0,)), ((), ())),
        preferred_element_type=jnp.float32)
    pos = jnp.concatenate(
        [pos_blocks[b] + off[b:b + 1, :] for b in range(nb)],
        axis=0)
    cnt_ref[...] = pos[_T - 1:_T, :]
    eidx_f = eidx.astype(jnp.float32)
    slot = eidx_f * _C + (pos - 1.0)
    slot = jnp.where(pos - 1.0 < _C, slot, float(_S))
    r16 = jax.lax.broadcasted_iota(jnp.int32, (_E, _E), 0)
    c16 = jax.lax.broadcasted_iota(jnp.int32, (_E, _E), 1)
    ltri16 = (r16 <= c16).astype(jnp.bfloat16)
    cl = jax.lax.dot_general(
        selb, ltri16, (((1,), (0,)), ((), ())),
        preferred_element_type=jnp.float32)
    low = sel * jnp.where(cl == 1.0, 1.0, 0.0)
    high = sel * jnp.where(cl == 2.0, 1.0, 0.0)
    dst0 = jnp.sum(slot * low, axis=1, keepdims=True)
    dst1 = jnp.sum(slot * high, axis=1, keepdims=True)
    w0 = jnp.sum(wfull * low, axis=1, keepdims=True)
    w1 = jnp.sum(wfull * high, axis=1, keepdims=True)
    meta_ref[:, 0:4] = jnp.concatenate([dst0, dst1, w0, w1], axis=1)


def _dot_t(a, b):
    return jax.lax.dot_general(
        a, b, (((1,), (1,)), ((), ())), preferred_element_type=jnp.float32)


def _group_body(cnt_ref, xg_ref, w1_ref, w3_ref, w2_ref, yg_ref):
    cb = pl.program_id(1)

    @pl.when(cb * _B < cnt_ref[0, pl.program_id(0)])
    def _():
        xgb = xg_ref[...].astype(jnp.bfloat16)
        g = _dot_t(xgb, w1_ref[0].astype(jnp.bfloat16))
        u = _dot_t(xgb, w3_ref[0].astype(jnp.bfloat16))
        h = (g * jax.nn.sigmoid(g) * u).astype(jnp.bfloat16)
        yg_ref[...] = _dot_t(h, w2_ref[0].astype(jnp.bfloat16))


def _dispatch_body(x_hbm, idx_hbm, xg_hbm, i_v, rows_v, semi, semr):
    c = jax.lax.axis_index("c")
    s = jax.lax.axis_index("s")
    wid = c * _NS + s
    j0 = wid * (2 * _T // _NW)
    r = j0 // _T
    t0 = j0 % _T

    def load(cc):
        tb = t0 + cc * 32
        ldi = pltpu.async_copy(
            idx_hbm.at[r, pl.ds(tb, 32)], i_v.at[cc % 2], semi)
        ldr = pltpu.async_copy(
            x_hbm.at[pl.ds(tb, 32)], rows_v.at[cc % 2], semr)
        return ldi, ldr

    pend = [load(0), load(1)]
    for cc in range(4):
        ldi, ldr = pend[cc % 2]
        ldi.wait()
        ldr.wait()
        pltpu.sync_copy(rows_v.at[cc % 2], xg_hbm.at[i_v.at[cc % 2]])
        if cc + 2 < 4:
            pend[cc % 2] = load(cc + 2)


def _combine_body(yg_hbm, idx_hbm, wexp_hbm, sh_hbm, out_hbm,
                  i_v, w_v, sh_v, y0_v, y1_v, sems):
    c = jax.lax.axis_index("c")
    s = jax.lax.axis_index("s")
    wid = c * _NS + s
    t0 = wid * (_T // _NW)

    pltpu.sync_copy(idx_hbm.at[0, pl.ds(t0, 64)], i_v.at[0])
    pltpu.sync_copy(idx_hbm.at[1, pl.ds(t0, 64)], i_v.at[1])
    pltpu.sync_copy(wexp_hbm.at[0, pl.ds(t0, 64)], w_v.at[0])
    pltpu.sync_copy(wexp_hbm.at[1, pl.ds(t0, 64)], w_v.at[1])
    for h in range(2):
        tb = t0 + h * 32
        g0 = pltpu.async_copy(yg_hbm.at[i_v.at[0, pl.ds(h * 32, 32)]],
                              y0_v, sems.at[0])
        g1 = pltpu.async_copy(yg_hbm.at[i_v.at[1, pl.ds(h * 32, 32)]],
                              y1_v, sems.at[1])
        pltpu.sync_copy(sh_hbm.at[pl.ds(tb, 32)], sh_v)
        g0.wait()
        g1.wait()

        def tok_body(i, carry):
            w0s = w_v[0, h * 32 + i]
            w1s = w_v[1, h * 32 + i]

            def col_body(k, carry2):
                sl = pl.ds(k * 16, 16)
                sh_v[i, sl] = (sh_v[i, sl] + w0s * y0_v[i, sl]
                               + w1s * y1_v[i, sl])
                return carry2

            return jax.lax.fori_loop(0, _D // 16, col_body, carry, unroll=4)

        jax.lax.fori_loop(0, 32, tok_body, 0)
        pltpu.sync_copy(sh_v, out_hbm.at[pl.ds(tb, 32)])


def _sc_mesh():
    return plsc.VectorSubcoreMesh(
        core_axis_name="c", subcore_axis_name="s",
        num_cores=_NC, num_subcores=_NS)


def _sc_dispatch(x, idx):
    return pl.kernel(
        _dispatch_body,
        out_type=jax.ShapeDtypeStruct((_XG_ROWS, _D), jnp.float32),
        mesh=_sc_mesh(),
        scratch_types=[pltpu.VMEM((2, 32), jnp.int32),
                       pltpu.VMEM((2, 32, _D), jnp.float32),
                       pltpu.SemaphoreType.DMA,
                       pltpu.SemaphoreType.DMA],
    )(x, idx)


def _sc_combine(yg, idx, wexp, shared):
    return pl.kernel(
        _combine_body,
        out_type=jax.ShapeDtypeStruct((_T, _D), jnp.float32),
        mesh=_sc_mesh(),
        scratch_types=[pltpu.VMEM((2, 64), jnp.int32),
                       pltpu.VMEM((2, 64, 16), jnp.float32),
                       pltpu.VMEM((32, _D), jnp.float32),
                       pltpu.VMEM((32, _D), jnp.float32),
                       pltpu.VMEM((32, _D), jnp.float32),
                       pltpu.SemaphoreType.DMA((2,))],
    )(yg, idx, wexp, shared)


def kernel(hidden_states, gate_w, expert_bias, w1, w3, w2, sw1, sw3, sw2):
    x = hidden_states.reshape(_T, _D)
    eb = expert_bias.reshape(1, _E)
    meta, cnts, shared = pl.pallas_call(
        _router_body,
        out_shape=(jax.ShapeDtypeStruct((_T, 128), jnp.float32),
                   jax.ShapeDtypeStruct((1, _E), jnp.float32),
                   jax.ShapeDtypeStruct((_T, _D), jnp.float32)),
    )(x, gate_w, eb, sw1, sw3, sw2)
    idx = jnp.transpose(meta[:, 0:2]).astype(jnp.int32)
    wexp = jnp.broadcast_to(
        jnp.transpose(meta[:, 2:4])[:, :, None], (2, _T, 16))
    counts = cnts.astype(jnp.int32)

    xg = _sc_dispatch(x, idx)

    yg = pl.pallas_call(
        _group_body,
        grid=(_E, _C // _B),
        in_specs=[
            pl.BlockSpec(memory_space=pltpu.SMEM),
            pl.BlockSpec((_B, _D), lambda e, cb: (e * (_C // _B) + cb, 0)),
            pl.BlockSpec((1, _DFF, _D), lambda e, cb: (e, 0, 0)),
            pl.BlockSpec((1, _DFF, _D), lambda e, cb: (e, 0, 0)),
            pl.BlockSpec((1, _D, _DFF), lambda e, cb: (e, 0, 0)),
        ],
        out_specs=pl.BlockSpec((_B, _D), lambda e, cb: (e * (_C // _B) + cb, 0)),
        out_shape=jax.ShapeDtypeStruct((_XG_ROWS, _D), jnp.float32),
    )(counts, xg, w1, w3, w2)

    out = _sc_combine(yg, idx, wexp, shared)
    return out


# speedup vs baseline: 3.1246x; 1.1904x over previous
import functools

import jax
import jax.numpy as jnp
from jax.experimental import pallas as pl
from jax.experimental.pallas import tpu as pltpu
from jax.experimental.pallas import tpu_sc as plsc

_T, _D, _E, _TOPK, _NG, _TG, _DFF, _DFFS = 2048, 1024, 16, 2, 4, 2, 512, 512
_GS = _E // _NG
_ROUTE_SCALE = 2.5
_C = 512
_B = 512
_S = _E * _C
_XG_ROWS = _S + _B
_NC, _NS = 2, 16
_NW = _NC * _NS


def _router_body(x_ref, gw_ref, eb_ref, sw1_ref, sw3_ref, sw2_ref,
                 meta_ref, cnt_ref, sh_ref):
    x = x_ref[...]
    xbv = x.astype(jnp.bfloat16)
    gsh = _dot_t(xbv, sw1_ref[...].astype(jnp.bfloat16))
    ush = _dot_t(xbv, sw3_ref[...].astype(jnp.bfloat16))
    hsh = (gsh * jax.nn.sigmoid(gsh) * ush).astype(jnp.bfloat16)
    sh_ref[...] = _dot_t(hsh, sw2_ref[...].astype(jnp.bfloat16))
    logits = jax.lax.dot_general(
        x, gw_ref[...], (((1,), (1,)), ((), ())),
        preferred_element_type=jnp.float32)
    scores = jax.nn.sigmoid(logits)
    sfc = scores + eb_ref[...]
    gs_cols = []
    for g in range(_NG):
        c = [sfc[:, g * _GS + i:g * _GS + i + 1] for i in range(_GS)]
        best = None
        for i in range(_GS):
            for j in range(i + 1, _GS):
                s = c[i] + c[j]
                best = s if best is None else jnp.maximum(best, s)
        gs_cols.append(best)
    gs = jnp.concatenate(gs_cols, axis=1)
    gidx = jax.lax.broadcasted_iota(jnp.int32, (_T, _NG), 1)
    grank = jnp.zeros((_T, _NG), jnp.float32)
    for j in range(_NG):
        gj = gs[:, j:j + 1]
        grank += jnp.where((gj > gs) | ((gj == gs) & (j < gidx)), 1.0, 0.0)
    gsel = (grank < _TG).astype(jnp.float32)
    emask = jnp.concatenate(
        [gsel[:, e // _GS:e // _GS + 1] for e in range(_E)], axis=1)
    tmp = sfc * emask
    eidx = jax.lax.broadcasted_iota(jnp.int32, (_T, _E), 1)
    erank = jnp.zeros((_T, _E), jnp.float32)
    for j in range(_E):
        vj = tmp[:, j:j + 1]
        erank += jnp.where((vj > tmp) | ((vj == tmp) & (j < eidx)), 1.0, 0.0)
    sel = jnp.where(erank < _TOPK, 1.0, 0.0)
    w = scores * sel
    denom = jnp.sum(w, axis=1, keepdims=True) + 1e-20
    wfull = w * (_ROUTE_SCALE / denom)
    selb = sel.astype(jnp.bfloat16)
    nb, bb = 16, _T // 16
    rb = jax.lax.broadcasted_iota(jnp.int32, (bb, 1), 0)
    cb2 = jax.lax.broadcasted_iota(jnp.int32, (1, bb), 1)
    l128 = (rb >= cb2).astype(jnp.bfloat16)
    pos_blocks = []
    tot_rows = []
    for b in range(nb):
        sb = selb[b * bb:(b + 1) * bb, :]
        pb = jax.lax.dot_general(
            l128, sb, (((1,), (0,)), ((), ())),
            preferred_element_type=jnp.float32)
        pos_blocks.append(pb)
        tot_rows.append(pb[bb - 1:bb, :])
    tot = jnp.concatenate(tot_rows, axis=0).astype(jnp.bfloat16)
    rnb = jax.lax.broadcasted_iota(jnp.int32, (nb, nb), 0)
    cnb = jax.lax.broadcasted_iota(jnp.int32, (nb, nb), 1)
    lstrict = (rnb > cnb).astype(jnp.bfloat16)
    off = jax.lax.dot_general(
        lstrict, tot, (((1,), (0,)), ((), ())),
        preferred_element_type=jnp.float32)
    pos = jnp.concatenate(
        [pos_blocks[b] + off[b:b + 1, :] for b in range(nb)],
        axis=0)
    cnt_ref[...] = pos[_T - 1:_T, :]
    eidx_f = eidx.astype(jnp.float32)
    slot = eidx_f * _C + (pos - 1.0)
    slot = jnp.where(pos - 1.0 < _C, slot, float(_S))
    r16 = jax.lax.broadcasted_iota(jnp.int32, (_E, _E), 0)
    c16 = jax.lax.broadcasted_iota(jnp.int32, (_E, _E), 1)
    ltri16 = (r16 <= c16).astype(jnp.bfloat16)
    cl = jax.lax.dot_general(
        selb, ltri16, (((1,), (0,)), ((), ())),
        preferred_element_type=jnp.float32)
    low = sel * jnp.where(cl == 1.0, 1.0, 0.0)
    high = sel * jnp.where(cl == 2.0, 1.0, 0.0)
    dst0 = jnp.sum(slot * low, axis=1, keepdims=True)
    dst1 = jnp.sum(slot * high, axis=1, keepdims=True)
    w0 = jnp.sum(wfull * low, axis=1, keepdims=True)
    w1 = jnp.sum(wfull * high, axis=1, keepdims=True)
    meta_ref[:, 0:4] = jnp.concatenate([dst0, dst1, w0, w1], axis=1)


def _dot_t(a, b):
    return jax.lax.dot_general(
        a, b, (((1,), (1,)), ((), ())), preferred_element_type=jnp.float32)


def _group_body(cnt_ref, xg_ref, w1_ref, w3_ref, w2_ref, yg_ref):
    cb = pl.program_id(1)

    @pl.when(cb * _B < cnt_ref[0, pl.program_id(0)])
    def _():
        xgb = xg_ref[...].astype(jnp.bfloat16)
        g = _dot_t(xgb, w1_ref[0].astype(jnp.bfloat16))
        u = _dot_t(xgb, w3_ref[0].astype(jnp.bfloat16))
        h = (g * jax.nn.sigmoid(g) * u).astype(jnp.bfloat16)
        yg_ref[...] = _dot_t(h, w2_ref[0].astype(jnp.bfloat16))


def _dispatch_body(x_hbm, idx_hbm, xg_hbm, i_v, rows_v, semi, semr):
    c = jax.lax.axis_index("c")
    s = jax.lax.axis_index("s")
    wid = c * _NS + s
    j0 = wid * (2 * _T // _NW)
    r = j0 // _T
    t0 = j0 % _T

    def load(cc):
        tb = t0 + cc * 32
        ldi = pltpu.async_copy(
            idx_hbm.at[r, pl.ds(tb, 32)], i_v.at[cc % 2], semi)
        ldr = pltpu.async_copy(
            x_hbm.at[pl.ds(tb, 32)], rows_v.at[cc % 2], semr)
        return ldi, ldr

    pend = [load(0), load(1)]
    for cc in range(4):
        ldi, ldr = pend[cc % 2]
        ldi.wait()
        ldr.wait()
        pltpu.sync_copy(rows_v.at[cc % 2], xg_hbm.at[i_v.at[cc % 2]])
        if cc + 2 < 4:
            pend[cc % 2] = load(cc + 2)


def _combine_body(yg_hbm, idx_hbm, wexp_hbm, sh_hbm, out_hbm,
                  i_v, w_v, sh_v, y0_v, y1_v, sems):
    c = jax.lax.axis_index("c")
    s = jax.lax.axis_index("s")
    wid = c * _NS + s
    t0 = wid * (_T // _NW)

    pltpu.sync_copy(idx_hbm.at[0, pl.ds(t0, 64)], i_v.at[0])
    pltpu.sync_copy(idx_hbm.at[1, pl.ds(t0, 64)], i_v.at[1])
    pltpu.sync_copy(wexp_hbm.at[0, pl.ds(t0, 64)], w_v.at[0])
    pltpu.sync_copy(wexp_hbm.at[1, pl.ds(t0, 64)], w_v.at[1])
    for h in range(2):
        tb = t0 + h * 32
        g0 = pltpu.async_copy(yg_hbm.at[i_v.at[0, pl.ds(h * 32, 32)]],
                              y0_v, sems.at[0])
        g1 = pltpu.async_copy(yg_hbm.at[i_v.at[1, pl.ds(h * 32, 32)]],
                              y1_v, sems.at[1])
        pltpu.sync_copy(sh_hbm.at[pl.ds(tb, 32)], sh_v)
        g0.wait()
        g1.wait()

        def tok_body(i, carry):
            w0s = w_v[0, h * 32 + i]
            w1s = w_v[1, h * 32 + i]

            def col_body(k, carry2):
                sl = pl.ds(k * 16, 16)
                sh_v[i, sl] = (sh_v[i, sl] + w0s * y0_v[i, sl]
                               + w1s * y1_v[i, sl])
                return carry2

            return jax.lax.fori_loop(0, _D // 16, col_body, carry, unroll=4)

        jax.lax.fori_loop(0, 32, tok_body, 0)
        pltpu.sync_copy(sh_v, out_hbm.at[pl.ds(tb, 32)])


def _sc_mesh():
    return plsc.VectorSubcoreMesh(
        core_axis_name="c", subcore_axis_name="s",
        num_cores=_NC, num_subcores=_NS)


def _sc_dispatch(x, idx):
    return pl.kernel(
        _dispatch_body,
        out_type=jax.ShapeDtypeStruct((_XG_ROWS, _D), jnp.float32),
        mesh=_sc_mesh(),
        scratch_types=[pltpu.VMEM((2, 32), jnp.int32),
                       pltpu.VMEM((2, 32, _D), jnp.float32),
                       pltpu.SemaphoreType.DMA,
                       pltpu.SemaphoreType.DMA],
    )(x, idx)


def _sc_combine(yg, idx, wexp, shared):
    return pl.kernel(
        _combine_body,
        out_type=jax.ShapeDtypeStruct((_T, _D), jnp.float32),
        mesh=_sc_mesh(),
        scratch_types=[pltpu.VMEM((2, 64), jnp.int32),
                       pltpu.VMEM((2, 64, 16), jnp.float32),
                       pltpu.VMEM((32, _D), jnp.float32),
                       pltpu.VMEM((32, _D), jnp.float32),
                       pltpu.VMEM((32, _D), jnp.float32),
                       pltpu.SemaphoreType.DMA((2,))],
    )(yg, idx, wexp, shared)


def kernel(hidden_states, gate_w, expert_bias, w1, w3, w2, sw1, sw3, sw2):
    x = hidden_states.reshape(_T, _D)
    eb = expert_bias.reshape(1, _E)
    meta, cnts, shared = pl.pallas_call(
        _router_body,
        out_shape=(jax.ShapeDtypeStruct((_T, 128), jnp.float32),
                   jax.ShapeDtypeStruct((1, _E), jnp.float32),
                   jax.ShapeDtypeStruct((_T, _D), jnp.float32)),
    )(x, gate_w, eb, sw1, sw3, sw2)
    idx = jnp.transpose(meta[:, 0:2]).astype(jnp.int32)
    wexp = jnp.broadcast_to(
        jnp.transpose(meta[:, 2:4])[:, :, None], (2, _T, 16))
    counts = cnts.astype(jnp.int32)

    xg = _sc_dispatch(x, idx)

    yg = pl.pallas_call(
        _group_body,
        grid=(_E, _C // _B),
        in_specs=[
            pl.BlockSpec(memory_space=pltpu.SMEM),
            pl.BlockSpec((_B, _D), lambda e, cb: (e * (_C // _B) + cb, 0)),
            pl.BlockSpec((1, _DFF, _D), lambda e, cb: (e, 0, 0)),
            pl.BlockSpec((1, _DFF, _D), lambda e, cb: (e, 0, 0)),
            pl.BlockSpec((1, _D, _DFF), lambda e, cb: (e, 0, 0)),
        ],
        out_specs=pl.BlockSpec((_B, _D), lambda e, cb: (e * (_C // _B) + cb, 0)),
        out_shape=jax.ShapeDtypeStruct((_XG_ROWS, _D), jnp.float32),
    )(counts, xg, w1, w3, w2)

    out = _sc_combine(yg, idx, wexp, shared)
    return out
